# baseline (speedup 1.0000x reference)
import functools

import jax
import jax.numpy as jnp
from jax import lax
from jax.experimental import pallas as pl
from jax.experimental.pallas import tpu as pltpu
from jax.experimental.pallas import tpu_sc as plsc

_NUM_WORKERS = 32
_LANES = 16
_TW = 128
_H = 8
_SEC = _H * _TW


def _make_kernel(batch, embed_dim):
    bpw = batch // _NUM_WORKERS
    nblk = bpw // _LANES
    mesh = plsc.VectorSubcoreMesh(core_axis_name="c", subcore_axis_name="s")

    @functools.partial(
        pl.kernel,
        mesh=mesh,
        compiler_params=pltpu.CompilerParams(
            needs_layout_passes=False, use_tc_tiling_on_sc=True),
        out_type=jax.ShapeDtypeStruct((batch,), jnp.float32),
        scratch_types=[
            pltpu.VMEM((bpw + _LANES,), jnp.int32),
            pltpu.VMEM((bpw + _LANES,), jnp.int32),
            pltpu.VMEM((embed_dim, 2 * _SEC), jnp.float32),
            pltpu.VMEM((bpw * embed_dim,), jnp.float32),
            pltpu.VMEM((bpw,), jnp.float32),
            pltpu.SemaphoreType.DMA,
            pltpu.SemaphoreType.DMA,
        ],
    )
    def scores_kernel(uids_hbm, iids_hbm, utab_hbm, itab_hbm, out_hbm,
                      uidx, iidx, ring, ucols, outv, sem_a, sem_b):
        wid = lax.axis_index("s") * 2 + lax.axis_index("c")
        base = wid * bpw
        pltpu.sync_copy(uids_hbm.at[pl.ds(base, bpw)], uidx.at[pl.ds(0, bpw)])
        pltpu.sync_copy(iids_hbm.at[pl.ds(base, bpw)], iidx.at[pl.ds(0, bpw)])
        uidx[pl.ds(bpw, _LANES)] = jnp.zeros((_LANES,), jnp.int32)
        iidx[pl.ds(bpw, _LANES)] = jnp.zeros((_LANES,), jnp.int32)
        lane = lax.iota(jnp.int32, _LANES)
        dims_lo = lax.iota(jnp.int32, _LANES)
        dims_hi = dims_lo + _LANES

        def fire_half(tab_hbm, vec, k0, sec, sem):
            for k in range(k0, k0 + _H):
                rt = pl.multiple_of((vec[k] // _TW) * _TW, _TW)
                pltpu.async_copy(
                    tab_hbm.at[:, pl.ds(rt, _TW)],
                    ring.at[:, pl.ds(sec + (k - k0) * _TW, _TW)], sem)

        def drain(tab_hbm, sec, sem):
            pltpu.make_async_copy(
                tab_hbm.at[:, pl.ds(0, _SEC)],
                ring.at[:, pl.ds(sec, _SEC)], sem).wait()

        def col_pair(vec, k, k0, sec):
            cols = jnp.zeros((_LANES,), jnp.int32) + (
                sec + (k - k0) * _TW + vec[k] % _TW)
            return (plsc.load_gather(ring, [dims_lo, cols]),
                    plsc.load_gather(ring, [dims_hi, cols]))

        vec0 = uidx[pl.ds(0, _LANES)]
        fire_half(utab_hbm, vec0, 0, 0, sem_a)

        def ublock(j16, carry):
            vec = uidx[pl.ds(j16 * _LANES, _LANES)]
            fire_half(utab_hbm, vec, _H, _SEC, sem_b)
            drain(utab_hbm, 0, sem_a)
            for k in range(_H):
                off = (j16 * _LANES + k) * embed_dim
                lo, hi = col_pair(vec, k, 0, 0)
                ucols[pl.ds(off, _LANES)] = lo
                ucols[pl.ds(off + _LANES, _LANES)] = hi
            vec_n = uidx[pl.ds(j16 * _LANES + _LANES, _LANES)]
            fire_half(utab_hbm, vec_n, 0, 0, sem_a)
            drain(utab_hbm, _SEC, sem_b)
            for k in range(_H, _LANES):
                off = (j16 * _LANES + k) * embed_dim
                lo, hi = col_pair(vec, k, _H, _SEC)
                ucols[pl.ds(off, _LANES)] = lo
                ucols[pl.ds(off + _LANES, _LANES)] = hi
            return carry

        lax.fori_loop(0, nblk, ublock, 0, unroll=False)
        drain(utab_hbm, 0, sem_a)

        vec0 = iidx[pl.ds(0, _LANES)]
        fire_half(itab_hbm, vec0, 0, 0, sem_a)

        def vblock(j16, carry):
            vec = iidx[pl.ds(j16 * _LANES, _LANES)]
            fire_half(itab_hbm, vec, _H, _SEC, sem_b)
            acc = jnp.zeros((_LANES,), jnp.float32)
            drain(itab_hbm, 0, sem_a)
            for k in range(_H):
                off = (j16 * _LANES + k) * embed_dim
                v_lo, v_hi = col_pair(vec, k, 0, 0)
                u_lo = ucols[pl.ds(off, _LANES)]
                u_hi = ucols[pl.ds(off + _LANES, _LANES)]
                s = lax.reduce_sum(u_lo * v_lo + u_hi * v_hi, axes=(0,))
                acc = jnp.where(lane == k, s, acc)
            vec_n = iidx[pl.ds(j16 * _LANES + _LANES, _LANES)]
            fire_half(itab_hbm, vec_n, 0, 0, sem_a)
            drain(itab_hbm, _SEC, sem_b)
            for k in range(_H, _LANES):
                off = (j16 * _LANES + k) * embed_dim
                v_lo, v_hi = col_pair(vec, k, _H, _SEC)
                u_lo = ucols[pl.ds(off, _LANES)]
                u_hi = ucols[pl.ds(off + _LANES, _LANES)]
                s = lax.reduce_sum(u_lo * v_lo + u_hi * v_hi, axes=(0,))
                acc = jnp.where(lane == k, s, acc)
            outv[pl.ds(j16 * _LANES, _LANES)] = acc
            return carry

        lax.fori_loop(0, nblk, vblock, 0, unroll=False)
        drain(itab_hbm, 0, sem_a)
        pltpu.sync_copy(outv, out_hbm.at[pl.ds(base, bpw)])

    return scores_kernel


@jax.jit
def kernel(user_ids, item_ids, user_table, item_table):
    batch = user_ids.shape[0]
    embed_dim = user_table.shape[1]
    uids = user_ids.astype(jnp.int32)
    iids = item_ids.astype(jnp.int32)
    utab_t = user_table.astype(jnp.float32).T
    itab_t = item_table.astype(jnp.float32).T
    return _make_kernel(batch, embed_dim)(uids, iids, utab_t, itab_t)


# speedup vs baseline: 1.0470x; 1.0470x over previous
import functools

import jax
import jax.numpy as jnp
from jax import lax
from jax.experimental import pallas as pl
from jax.experimental.pallas import tpu as pltpu
from jax.experimental.pallas import tpu_sc as plsc

_NUM_WORKERS = 32
_LANES = 16
_TW = 128


def _make_kernel(batch, embed_dim):
    bpw = batch // _NUM_WORKERS
    nblk = bpw // _LANES
    mesh = plsc.VectorSubcoreMesh(core_axis_name="c", subcore_axis_name="s")

    @functools.partial(
        pl.kernel,
        mesh=mesh,
        compiler_params=pltpu.CompilerParams(
            needs_layout_passes=False, use_tc_tiling_on_sc=True),
        out_type=jax.ShapeDtypeStruct((batch,), jnp.float32),
        scratch_types=[
            pltpu.VMEM((bpw,), jnp.int32),
            pltpu.VMEM((bpw,), jnp.int32),
            pltpu.VMEM((embed_dim, _LANES * _TW), jnp.float32),
            pltpu.VMEM((bpw * embed_dim,), jnp.float32),
            pltpu.VMEM((bpw,), jnp.float32),
            pltpu.SemaphoreType.DMA,
        ],
    )
    def scores_kernel(uids_hbm, iids_hbm, utab_hbm, itab_hbm, out_hbm,
                      uidx, iidx, ring, ucols, outv, sem):
        wid = lax.axis_index("s") * 2 + lax.axis_index("c")
        base = wid * bpw
        pltpu.sync_copy(uids_hbm.at[pl.ds(base, bpw)], uidx)
        pltpu.sync_copy(iids_hbm.at[pl.ds(base, bpw)], iidx)
        lane = lax.iota(jnp.int32, _LANES)
        dims_lo = lax.iota(jnp.int32, _LANES)
        dims_hi = dims_lo + _LANES

        def fetch_block(tab_hbm, idx_ref, j16):
            vec = idx_ref[pl.ds(j16 * _LANES, _LANES)]
            for k in range(_LANES):
                rt = pl.multiple_of((vec[k] // _TW) * _TW, _TW)
                pltpu.async_copy(
                    tab_hbm.at[:, pl.ds(rt, _TW)],
                    ring.at[:, pl.ds(k * _TW, _TW)], sem)
            pltpu.make_async_copy(
                tab_hbm.at[:, pl.ds(0, _LANES * _TW)], ring, sem).wait()
            return vec

        def col_pair(vec, k):
            cols = jnp.zeros((_LANES,), jnp.int32) + (k * _TW + vec[k] % _TW)
            return (plsc.load_gather(ring, [dims_lo, cols]),
                    plsc.load_gather(ring, [dims_hi, cols]))

        def ublock(j16, carry):
            vec = fetch_block(utab_hbm, uidx, j16)
            for k in range(_LANES):
                off = (j16 * _LANES + k) * embed_dim
                lo, hi = col_pair(vec, k)
                ucols[pl.ds(off, _LANES)] = lo
                ucols[pl.ds(off + _LANES, _LANES)] = hi
            return carry

        lax.fori_loop(0, nblk, ublock, 0, unroll=False)

        def vblock(j16, carry):
            vec = fetch_block(itab_hbm, iidx, j16)
            acc = jnp.zeros((_LANES,), jnp.float32)
            for k in range(_LANES):
                off = (j16 * _LANES + k) * embed_dim
                v_lo, v_hi = col_pair(vec, k)
                u_lo = ucols[pl.ds(off, _LANES)]
                u_hi = ucols[pl.ds(off + _LANES, _LANES)]
                s = lax.reduce_sum(u_lo * v_lo + u_hi * v_hi, axes=(0,))
                acc = jnp.where(lane == k, s, acc)
            outv[pl.ds(j16 * _LANES, _LANES)] = acc
            return carry

        lax.fori_loop(0, nblk, vblock, 0, unroll=False)
        pltpu.sync_copy(outv, out_hbm.at[pl.ds(base, bpw)])

    return scores_kernel


@jax.jit
def kernel(user_ids, item_ids, user_table, item_table):
    batch = user_ids.shape[0]
    embed_dim = user_table.shape[1]
    uids = user_ids.astype(jnp.int32)
    iids = item_ids.astype(jnp.int32)
    utab_t = user_table.astype(jnp.float32).T
    itab_t = item_table.astype(jnp.float32).T
    return _make_kernel(batch, embed_dim)(uids, iids, utab_t, itab_t)


# speedup vs baseline: 1.1110x; 1.0612x over previous
import functools

import jax
import jax.numpy as jnp
from jax import lax
from jax.experimental import pallas as pl
from jax.experimental.pallas import tpu as pltpu
from jax.experimental.pallas import tpu_sc as plsc

_NUM_WORKERS = 32
_LANES = 16
_TW = 128


def _make_kernel(batch, embed_dim):
    bpw = batch // _NUM_WORKERS
    nblk = bpw // _LANES
    mesh = plsc.VectorSubcoreMesh(core_axis_name="c", subcore_axis_name="s")

    @functools.partial(
        pl.kernel,
        mesh=mesh,
        compiler_params=pltpu.CompilerParams(
            needs_layout_passes=False, use_tc_tiling_on_sc=True),
        out_type=jax.ShapeDtypeStruct((batch,), jnp.float32),
        scratch_types=[
            pltpu.VMEM((bpw,), jnp.int32),
            pltpu.VMEM((bpw,), jnp.int32),
            pltpu.VMEM((embed_dim, _LANES * _TW), jnp.float32),
            pltpu.VMEM((bpw * embed_dim,), jnp.float32),
            pltpu.VMEM((bpw,), jnp.float32),
            pltpu.SemaphoreType.DMA,
        ],
    )
    def scores_kernel(uids_hbm, iids_hbm, utab_hbm, itab_hbm, out_hbm,
                      uidx, iidx, ring, ucols, outv, sem):
        wid = lax.axis_index("s") * 2 + lax.axis_index("c")
        base = wid * bpw
        pltpu.sync_copy(uids_hbm.at[pl.ds(base, bpw)], uidx)
        pltpu.sync_copy(iids_hbm.at[pl.ds(base, bpw)], iidx)
        lane = lax.iota(jnp.int32, _LANES)

        def fetch_block(tab_hbm, idx_ref, j16):
            vec = idx_ref[pl.ds(j16 * _LANES, _LANES)]
            for k in range(_LANES):
                rt = pl.multiple_of((vec[k] // _TW) * _TW, _TW)
                pltpu.async_copy(
                    tab_hbm.at[:, pl.ds(rt, _TW)],
                    ring.at[:, pl.ds(k * _TW, _TW)], sem)
            pltpu.make_async_copy(
                tab_hbm.at[:, pl.ds(0, _LANES * _TW)], ring, sem).wait()
            return lane * _TW + vec % _TW

        def ublock(j16, carry):
            cols = fetch_block(utab_hbm, uidx, j16)
            for d in range(embed_dim):
                u_d = plsc.load_gather(
                    ring, [jnp.zeros((_LANES,), jnp.int32) + d, cols])
                ucols[pl.ds((j16 * embed_dim + d) * _LANES, _LANES)] = u_d
            return carry

        lax.fori_loop(0, nblk, ublock, 0, unroll=False)

        def vblock(j16, carry):
            cols = fetch_block(itab_hbm, iidx, j16)
            acc = jnp.zeros((_LANES,), jnp.float32)
            for d in range(embed_dim):
                v_d = plsc.load_gather(
                    ring, [jnp.zeros((_LANES,), jnp.int32) + d, cols])
                u_d = ucols[pl.ds((j16 * embed_dim + d) * _LANES, _LANES)]
                acc = acc + u_d * v_d
            outv[pl.ds(j16 * _LANES, _LANES)] = acc
            return carry

        lax.fori_loop(0, nblk, vblock, 0, unroll=False)
        pltpu.sync_copy(outv, out_hbm.at[pl.ds(base, bpw)])

    return scores_kernel


@jax.jit
def kernel(user_ids, item_ids, user_table, item_table):
    batch = user_ids.shape[0]
    embed_dim = user_table.shape[1]
    uids = user_ids.astype(jnp.int32)
    iids = item_ids.astype(jnp.int32)
    utab_t = user_table.astype(jnp.float32).T
    itab_t = item_table.astype(jnp.float32).T
    return _make_kernel(batch, embed_dim)(uids, iids, utab_t, itab_t)
